# baseline (speedup 1.0000x reference)
import jax
import jax.numpy as jnp
from jax.experimental import pallas as pl


def _pos_kernel(row_ref, col_ref, out_ref):
    col = col_ref[:32, :]
    row = row_ref[:32, :]
    colT = col.T
    rowT = row.T
    out_ref[0, :256] = jnp.broadcast_to(colT[:, None, :], (256, 32, 32))
    out_ref[0, 256:] = jnp.broadcast_to(rowT[:, :, None], (256, 32, 32))


def kernel(mask, row_embed, col_embed):
    B, H, W = mask.shape
    C = row_embed.shape[1]
    out = pl.pallas_call(
        _pos_kernel,
        grid=(B,),
        in_specs=[
            pl.BlockSpec(row_embed.shape, lambda b: (0, 0)),
            pl.BlockSpec(col_embed.shape, lambda b: (0, 0)),
        ],
        out_specs=pl.BlockSpec((1, 2 * C, H, W), lambda b: (b, 0, 0, 0)),
        out_shape=jax.ShapeDtypeStruct((B, 2 * C, H, W), jnp.float32),
    )(row_embed, col_embed)
    return out


# speedup vs baseline: 2.5880x; 2.5880x over previous
import jax
import jax.numpy as jnp
from jax import lax
from jax.experimental import pallas as pl


def _pos_kernel(row_ref, col_ref, out_ref):
    col = col_ref[:32, :]
    row = row_ref[:32, :]
    k = lax.broadcasted_iota(jnp.int32, (32, 1024), 1)
    src = lax.broadcasted_iota(jnp.int32, (32, 1024), 0)
    sel_w = (k % 32 == src).astype(jnp.float32)
    sel_h = (k // 32 == src).astype(jnp.float32)
    dn = (((0,), (0,)), ((), ()))
    plane_col = lax.dot_general(col, sel_w, dn, preferred_element_type=jnp.float32)
    plane_row = lax.dot_general(row, sel_h, dn, preferred_element_type=jnp.float32)
    out_ref[0, :256] = plane_col
    out_ref[0, 256:] = plane_row


def kernel(mask, row_embed, col_embed):
    B, H, W = mask.shape
    C = row_embed.shape[1]
    out = pl.pallas_call(
        _pos_kernel,
        grid=(B,),
        in_specs=[
            pl.BlockSpec(row_embed.shape, lambda b: (0, 0)),
            pl.BlockSpec(col_embed.shape, lambda b: (0, 0)),
        ],
        out_specs=pl.BlockSpec((1, 2 * C, H * W), lambda b: (b, 0, 0)),
        out_shape=jax.ShapeDtypeStruct((B, 2 * C, H * W), jnp.float32),
    )(row_embed, col_embed)
    return out.reshape(B, 2 * C, H, W)


# speedup vs baseline: 2.8091x; 1.0854x over previous
import jax
import jax.numpy as jnp
from jax import lax
from jax.experimental import pallas as pl
from jax.experimental.pallas import tpu as pltpu

_B, _C2, _HW = 16, 512, 1024


def _pos_kernel(row_ref, col_ref, out_ref, plane, sem):
    col = col_ref[:32, :]
    row = row_ref[:32, :]
    k = lax.broadcasted_iota(jnp.int32, (32, _HW), 1)
    src = lax.broadcasted_iota(jnp.int32, (32, _HW), 0)
    sel_w = (k % 32 == src).astype(jnp.float32)
    sel_h = (k // 32 == src).astype(jnp.float32)
    dn = (((0,), (0,)), ((), ()))
    plane[:256] = lax.dot_general(col, sel_w, dn, preferred_element_type=jnp.float32)
    plane[256:] = lax.dot_general(row, sel_h, dn, preferred_element_type=jnp.float32)
    copies = [pltpu.make_async_copy(plane, out_ref.at[b], sem) for b in range(_B)]
    for c in copies:
        c.start()
    for c in copies:
        c.wait()


def kernel(mask, row_embed, col_embed):
    B, H, W = mask.shape
    C = row_embed.shape[1]
    out = pl.pallas_call(
        _pos_kernel,
        in_specs=[
            pl.BlockSpec(memory_space=pltpu.VMEM),
            pl.BlockSpec(memory_space=pltpu.VMEM),
        ],
        out_specs=pl.BlockSpec(memory_space=pl.ANY),
        out_shape=jax.ShapeDtypeStruct((B, 2 * C, H * W), jnp.float32),
        scratch_shapes=[
            pltpu.VMEM((2 * C, H * W), jnp.float32),
            pltpu.SemaphoreType.DMA,
        ],
    )(row_embed, col_embed)
    return out.reshape(B, 2 * C, H, W)
